# Initial kernel scaffold; baseline (speedup 1.0000x reference)
import functools

import jax
import jax.numpy as jnp
from jax import lax
from jax.experimental import pallas as pl
from jax.experimental.pallas import tpu as pltpu
from jax.experimental.pallas import tpu_sc as plsc

_NC = 2
_NS = 16
_NW = _NC * _NS

_VOCAB = 1000000
_DIM = 32
_B = 16384 * 200

_CHUNK = 2048
_PER_W = _B // _NW
_NCHUNK = _PER_W // _CHUNK


def _make_kernel():
  mesh = plsc.VectorSubcoreMesh(
      core_axis_name="c", subcore_axis_name="s",
      num_cores=_NC, num_subcores=_NS)

  @functools.partial(
      pl.kernel,
      out_type=jax.ShapeDtypeStruct((_B, _DIM), jnp.float32),
      mesh=mesh,
      scratch_types=[
          pltpu.VMEM((_CHUNK,), jnp.int32),
          pltpu.VMEM((_CHUNK, _DIM), jnp.float32),
          pltpu.SemaphoreType.DMA,
      ],
  )
  def gather_kernel(x_hbm, table_hbm, out_hbm, idx_v, rows_v, sem):
    wid = lax.axis_index("s") * _NC + lax.axis_index("c")
    base = wid * _PER_W

    def chunk_body(i, carry):
      start = base + i * _CHUNK
      pltpu.sync_copy(x_hbm.at[pl.ds(start, _CHUNK)], idx_v)
      pltpu.async_copy(table_hbm.at[idx_v], rows_v, sem).wait()
      pltpu.sync_copy(rows_v, out_hbm.at[pl.ds(start, _CHUNK)])
      return carry

    lax.fori_loop(0, _NCHUNK, chunk_body, 0)

  return gather_kernel


_GATHER = _make_kernel()


def kernel(x, table):
  flat = x.reshape(-1).astype(jnp.int32)
  out = _GATHER(flat, table)
  return out.reshape(x.shape + (_DIM,))


# speedup vs baseline: 4.9513x; 4.9513x over previous
import functools

import jax
import jax.numpy as jnp
from jax import lax
from jax.experimental import pallas as pl
from jax.experimental.pallas import tpu as pltpu
from jax.experimental.pallas import tpu_sc as plsc

_NC = 2
_NS = 16
_NW = _NC * _NS

_VOCAB = 1000000
_DIM = 32
_B = 16384 * 200

_CHUNK = 2048
_PER_W = _B // _NW
_NCHUNK = _PER_W // _CHUNK


def _make_kernel():
  mesh = plsc.VectorSubcoreMesh(
      core_axis_name="c", subcore_axis_name="s",
      num_cores=_NC, num_subcores=_NS)

  @functools.partial(
      pl.kernel,
      out_type=jax.ShapeDtypeStruct((_B, _DIM), jnp.float32),
      mesh=mesh,
      scratch_types=[
          pltpu.VMEM((_CHUNK,), jnp.int32),
          pltpu.VMEM((_CHUNK, _DIM), jnp.float32),
          pltpu.SemaphoreType.DMA,
      ],
      compiler_params=pltpu.CompilerParams(use_tc_tiling_on_sc=False),
  )
  def gather_kernel(x_hbm, table_hbm, out_hbm, idx_v, rows_v, sem):
    wid = lax.axis_index("s") * _NC + lax.axis_index("c")
    base = wid * _PER_W

    def chunk_body(i, carry):
      start = base + i * _CHUNK
      pltpu.sync_copy(x_hbm.at[pl.ds(start, _CHUNK)], idx_v)
      pltpu.async_copy(table_hbm.at[idx_v], rows_v, sem).wait()
      pltpu.sync_copy(rows_v, out_hbm.at[pl.ds(start, _CHUNK)])
      return carry

    lax.fori_loop(0, _NCHUNK, chunk_body, 0)

  return gather_kernel


_GATHER = _make_kernel()


def kernel(x, table):
  flat = x.reshape(-1).astype(jnp.int32)
  out = _GATHER(flat, table)
  return out.reshape(x.shape + (_DIM,))


# speedup vs baseline: 5.0519x; 1.0203x over previous
import functools

import jax
import jax.numpy as jnp
from jax import lax
from jax.experimental import pallas as pl
from jax.experimental.pallas import tpu as pltpu
from jax.experimental.pallas import tpu_sc as plsc

_NC = 2
_NS = 16
_NW = _NC * _NS

_DIM = 32
_B = 16384 * 200

_CHUNK = 800
_PER_W = _B // _NW
_NCHUNK = _PER_W // _CHUNK
_DEPTH = 4


def _make_kernel():
  mesh = plsc.VectorSubcoreMesh(
      core_axis_name="c", subcore_axis_name="s",
      num_cores=_NC, num_subcores=_NS)

  @functools.partial(
      pl.kernel,
      out_type=jax.ShapeDtypeStruct((_B, _DIM), jnp.float32),
      mesh=mesh,
      scratch_types=(
          [pltpu.VMEM((_CHUNK,), jnp.int32) for _ in range(_DEPTH)]
          + [pltpu.VMEM((_CHUNK, _DIM), jnp.float32) for _ in range(_DEPTH)]
          + [pltpu.SemaphoreType.DMA] * 3
      ),
      compiler_params=pltpu.CompilerParams(use_tc_tiling_on_sc=False),
  )
  def gather_kernel(x_hbm, table_hbm, out_hbm, *scratch):
    idx_bufs = scratch[:_DEPTH]
    row_bufs = scratch[_DEPTH:2 * _DEPTH]
    sem_i, sem_g, sem_s = scratch[2 * _DEPTH:]

    wid = lax.axis_index("s") * _NC + lax.axis_index("c")
    base = wid * _PER_W

    def issue_idx(i, u):
      pltpu.async_copy(
          x_hbm.at[pl.ds(base + i * _CHUNK, _CHUNK)], idx_bufs[u], sem_i)

    def issue_gather(u):
      pltpu.async_copy(table_hbm.at[idx_bufs[u]], row_bufs[u], sem_g)

    def issue_store(i, u):
      pltpu.async_copy(
          row_bufs[u], out_hbm.at[pl.ds(base + i * _CHUNK, _CHUNK)], sem_s)

    def wait_idx():
      pltpu.make_async_copy(
          x_hbm.at[pl.ds(0, _CHUNK)], idx_bufs[0], sem_i).wait()

    def wait_gather():
      pltpu.make_async_copy(
          table_hbm.at[idx_bufs[0]], row_bufs[0], sem_g).wait()

    def wait_store():
      pltpu.make_async_copy(
          row_bufs[0], out_hbm.at[pl.ds(0, _CHUNK)], sem_s).wait()

    for u in range(_DEPTH):
      issue_idx(u, u)
    wait_idx()
    issue_gather(0)
    wait_idx()
    issue_gather(1)

    def round_body(r, carry):
      for u in range(_DEPTH):
        i = r * _DEPTH + u
        wait_gather()
        issue_store(i, u)

        @pl.when(i + _DEPTH < _NCHUNK)
        def _():
          issue_idx(i + _DEPTH, u)

        @pl.when((i >= 2) & (i + 2 < _NCHUNK))
        def _():
          wait_store()

        @pl.when(i + 2 < _NCHUNK)
        def _():
          wait_idx()
          issue_gather((u + 2) % _DEPTH)
      return carry

    lax.fori_loop(0, _NCHUNK // _DEPTH, round_body, 0)
    for _ in range(_DEPTH):
      wait_store()

  return gather_kernel


_GATHER = _make_kernel()


def kernel(x, table):
  flat = x.reshape(-1).astype(jnp.int32)
  out = _GATHER(flat, table)
  return out.reshape(x.shape + (_DIM,))
